# baseline (device time: 26823 ns/iter reference)
import jax
import jax.numpy as jnp
from jax import lax
from jax.experimental import pallas as pl
from jax.experimental.pallas import tpu as pltpu

N_DEV = 32
N_EXP = 128
CAP = 6
EXP_PER = N_EXP // N_DEV
SLOTS = EXP_PER * CAP
N_TOK = 1024
TOK_PER = N_TOK // N_DEV


N_STAGES = N_DEV.bit_length() - 1


def _rd_allgather(c_block):
    slots, h = c_block.shape

    def body(c_ref, out_ref, send_sems, recv_sems):
        my = lax.axis_index("i")

        barrier = pltpu.get_barrier_semaphore()
        for k in range(N_STAGES):
            pl.semaphore_signal(
                barrier, inc=1,
                device_id=(my ^ (1 << k),),
                device_id_type=pl.DeviceIdType.MESH,
            )
        pl.semaphore_wait(barrier, N_STAGES)

        out_ref[pl.ds(my * slots, slots), :] = c_ref[...]

        rdmas = []
        for k in range(N_STAGES):
            size = 1 << k
            base = (my // size) * size
            rdma = pltpu.make_async_remote_copy(
                src_ref=out_ref.at[pl.ds(base * slots, size * slots), :],
                dst_ref=out_ref.at[pl.ds(base * slots, size * slots), :],
                send_sem=send_sems.at[k],
                recv_sem=recv_sems.at[k],
                device_id=(my ^ size,),
                device_id_type=pl.DeviceIdType.MESH,
            )
            rdma.start()
            rdma.wait_recv()
            rdmas.append(rdma)
        for rdma in rdmas:
            rdma.wait_send()

    return pl.pallas_call(
        body,
        out_shape=jax.ShapeDtypeStruct((N_DEV * slots, h), c_block.dtype),
        in_specs=[pl.BlockSpec(memory_space=pltpu.VMEM)],
        out_specs=pl.BlockSpec(memory_space=pltpu.VMEM),
        scratch_shapes=[
            pltpu.SemaphoreType.DMA((N_STAGES,)),
            pltpu.SemaphoreType.DMA((N_STAGES,)),
        ],
        compiler_params=pltpu.CompilerParams(collective_id=0),
    )(c_block)


PAD = 8
SLOTS_P = EXP_PER * PAD
D = 512


def _direct_moe(xs, w, valid, dest_chip, dest_row, kept_m):
    h = w.shape[-1]

    def body(valid_ref, dchip_ref, drow_ref, kept_ref,
             xs_ref, w_ref, out_ref, c_ref, send_sems, recv_sems):
        my = lax.axis_index("i")

        out_ref[...] = jnp.zeros_like(out_ref)

        barrier = pltpu.get_barrier_semaphore()
        for off in range(1, N_DEV):
            pl.semaphore_signal(
                barrier, inc=1,
                device_id=((my + off) % N_DEV,),
                device_id_type=pl.DeviceIdType.MESH,
            )
        pl.semaphore_wait(barrier, N_DEV - 1)

        for le in range(EXP_PER):
            c_ref[pl.ds(le * PAD, PAD), :] = jnp.dot(
                xs_ref[pl.ds(le * PAD, PAD), :],
                w_ref[le],
                preferred_element_type=jnp.float32,
            )

        for s in range(SLOTS_P):
            @pl.when(valid_ref[s] != 0)
            def _send(s=s):
                row = drow_ref[s]
                pltpu.make_async_remote_copy(
                    src_ref=c_ref.at[pl.ds(s, 1), :],
                    dst_ref=out_ref.at[pl.ds(row, 1), :],
                    send_sem=send_sems.at[s],
                    recv_sem=recv_sems.at[row],
                    device_id=(dchip_ref[s],),
                    device_id_type=pl.DeviceIdType.MESH,
                ).start()

        for j in range(TOK_PER):
            @pl.when(kept_ref[j] != 0)
            def _recv(j=j):
                pltpu.make_async_remote_copy(
                    src_ref=c_ref.at[pl.ds(0, 1), :],
                    dst_ref=out_ref.at[pl.ds(j, 1), :],
                    send_sem=send_sems.at[0],
                    recv_sem=recv_sems.at[j],
                    device_id=(my,),
                    device_id_type=pl.DeviceIdType.MESH,
                ).wait_recv()

        for s in range(SLOTS_P):
            @pl.when(valid_ref[s] != 0)
            def _drain(s=s):
                pltpu.make_async_remote_copy(
                    src_ref=c_ref.at[pl.ds(s, 1), :],
                    dst_ref=out_ref.at[pl.ds(0, 1), :],
                    send_sem=send_sems.at[s],
                    recv_sem=recv_sems.at[0],
                    device_id=(my,),
                    device_id_type=pl.DeviceIdType.MESH,
                ).wait_send()

        for off in range(1, N_DEV):
            pl.semaphore_signal(
                barrier, inc=1,
                device_id=((my + off) % N_DEV,),
                device_id_type=pl.DeviceIdType.MESH,
            )
        pl.semaphore_wait(barrier, N_DEV - 1)

    return pl.pallas_call(
        body,
        out_shape=jax.ShapeDtypeStruct((TOK_PER, h), jnp.float32),
        in_specs=[
            pl.BlockSpec(memory_space=pltpu.SMEM),
            pl.BlockSpec(memory_space=pltpu.SMEM),
            pl.BlockSpec(memory_space=pltpu.SMEM),
            pl.BlockSpec(memory_space=pltpu.SMEM),
            pl.BlockSpec(memory_space=pltpu.VMEM),
            pl.BlockSpec(memory_space=pltpu.VMEM),
        ],
        out_specs=pl.BlockSpec(memory_space=pltpu.VMEM),
        scratch_shapes=[
            pltpu.VMEM((SLOTS_P, h), jnp.float32),
            pltpu.SemaphoreType.DMA((SLOTS_P,)),
            pltpu.SemaphoreType.DMA((TOK_PER,)),
        ],
        compiler_params=pltpu.CompilerParams(collective_id=0),
    )(valid, dest_chip, dest_row, kept_m, xs, w)


def kernel(x, router_W, route_idx, expert_W):
    del router_W
    me = lax.axis_index("i")

    e = route_idx[:, 0].astype(jnp.int32)
    ti = jnp.arange(N_TOK, dtype=jnp.int32)
    rank = ((e[None, :] == e[:, None]) & (ti[None, :] < ti[:, None])).sum(
        axis=1, dtype=jnp.int32
    )
    kept = rank < CAP

    e_loc = e - me * EXP_PER
    mine = kept & (e_loc >= 0) & (e_loc < EXP_PER)
    slot = e_loc * PAD + rank
    srange = jnp.arange(SLOTS_P, dtype=jnp.int32)
    hit = mine[None, :] & (slot[None, :] == srange[:, None])
    tok_of_slot = (hit * ti[None, :]).sum(axis=1, dtype=jnp.int32)
    valid = hit.sum(axis=1, dtype=jnp.int32)
    dest_chip = tok_of_slot // TOK_PER
    dest_row = tok_of_slot % TOK_PER

    t_mine = me * TOK_PER + jnp.arange(TOK_PER, dtype=jnp.int32)
    kept_m = kept[t_mine].astype(jnp.int32)

    xs = x[tok_of_slot]
    return _direct_moe(xs, expert_W, valid, dest_chip, dest_row, kept_m)


# device time: 25465 ns/iter; 1.0533x vs baseline; 1.0533x over previous
import jax
import jax.numpy as jnp
from jax import lax
from jax.experimental import pallas as pl
from jax.experimental.pallas import tpu as pltpu

N_DEV = 32
N_EXP = 128
CAP = 6
EXP_PER = N_EXP // N_DEV
N_TOK = 1024
TOK_PER = N_TOK // N_DEV
PAD = 8
SLOTS_P = EXP_PER * PAD
D = 512


def _direct_moe(xs, w, valid, dest_chip, dest_row, kept_m):
    h = w.shape[-1]

    def body(valid_ref, dchip_ref, drow_ref, kept_ref,
             xs_ref, w_ref, out_ref, c_ref, send_sems, recv_sems):
        my = lax.axis_index("i")

        out_ref[...] = jnp.zeros_like(out_ref)

        barrier = pltpu.get_barrier_semaphore()
        for off in range(1, N_DEV):
            pl.semaphore_signal(
                barrier, inc=1,
                device_id=((my + off) % N_DEV,),
                device_id_type=pl.DeviceIdType.MESH,
            )
        pl.semaphore_wait(barrier, N_DEV - 1)

        for le in range(EXP_PER):
            c_ref[pl.ds(le * PAD, PAD), :] = jnp.dot(
                xs_ref[pl.ds(le * PAD, PAD), :],
                w_ref[le],
                preferred_element_type=jnp.float32,
            )
            for s in range(le * PAD, (le + 1) * PAD):
                @pl.when(valid_ref[s] != 0)
                def _send(s=s):
                    row = drow_ref[s]
                    pltpu.make_async_remote_copy(
                        src_ref=c_ref.at[pl.ds(s, 1), :],
                        dst_ref=out_ref.at[pl.ds(row, 1), :],
                        send_sem=send_sems.at[s],
                        recv_sem=recv_sems.at[row],
                        device_id=(dchip_ref[s],),
                        device_id_type=pl.DeviceIdType.MESH,
                    ).start()

        for j in range(TOK_PER):
            @pl.when(kept_ref[j] != 0)
            def _recv(j=j):
                pltpu.make_async_remote_copy(
                    src_ref=c_ref.at[pl.ds(0, 1), :],
                    dst_ref=out_ref.at[pl.ds(j, 1), :],
                    send_sem=send_sems.at[0],
                    recv_sem=recv_sems.at[j],
                    device_id=(my,),
                    device_id_type=pl.DeviceIdType.MESH,
                ).wait_recv()

        for s in range(SLOTS_P):
            @pl.when(valid_ref[s] != 0)
            def _drain(s=s):
                pltpu.make_async_remote_copy(
                    src_ref=c_ref.at[pl.ds(s, 1), :],
                    dst_ref=out_ref.at[pl.ds(0, 1), :],
                    send_sem=send_sems.at[s],
                    recv_sem=recv_sems.at[0],
                    device_id=(my,),
                    device_id_type=pl.DeviceIdType.MESH,
                ).wait_send()

        for off in range(1, N_DEV):
            pl.semaphore_signal(
                barrier, inc=1,
                device_id=((my + off) % N_DEV,),
                device_id_type=pl.DeviceIdType.MESH,
            )
        pl.semaphore_wait(barrier, N_DEV - 1)

    return pl.pallas_call(
        body,
        out_shape=jax.ShapeDtypeStruct((TOK_PER, h), jnp.float32),
        in_specs=[
            pl.BlockSpec(memory_space=pltpu.SMEM),
            pl.BlockSpec(memory_space=pltpu.SMEM),
            pl.BlockSpec(memory_space=pltpu.SMEM),
            pl.BlockSpec(memory_space=pltpu.SMEM),
            pl.BlockSpec(memory_space=pltpu.VMEM),
            pl.BlockSpec(memory_space=pltpu.VMEM),
        ],
        out_specs=pl.BlockSpec(memory_space=pltpu.VMEM),
        scratch_shapes=[
            pltpu.VMEM((SLOTS_P, h), jnp.float32),
            pltpu.SemaphoreType.DMA((SLOTS_P,)),
            pltpu.SemaphoreType.DMA((TOK_PER,)),
        ],
        compiler_params=pltpu.CompilerParams(collective_id=0),
    )(valid, dest_chip, dest_row, kept_m, xs, w)


def kernel(x, router_W, route_idx, expert_W):
    del router_W
    me = lax.axis_index("i")

    e = route_idx[:, 0].astype(jnp.int32)
    ti = jnp.arange(N_TOK, dtype=jnp.int32)
    rank = ((e[None, :] == e[:, None]) & (ti[None, :] < ti[:, None])).sum(
        axis=1, dtype=jnp.int32
    )
    kept = rank < CAP

    e_loc = e - me * EXP_PER
    mine = kept & (e_loc >= 0) & (e_loc < EXP_PER)
    slot = e_loc * PAD + rank
    srange = jnp.arange(SLOTS_P, dtype=jnp.int32)
    hit = mine[None, :] & (slot[None, :] == srange[:, None])
    tok_of_slot = (hit * ti[None, :]).sum(axis=1, dtype=jnp.int32)
    valid = hit.sum(axis=1, dtype=jnp.int32)
    dest_chip = tok_of_slot // TOK_PER
    dest_row = tok_of_slot % TOK_PER

    t_mine = me * TOK_PER + jnp.arange(TOK_PER, dtype=jnp.int32)
    kept_m = kept[t_mine].astype(jnp.int32)

    xs = x[tok_of_slot]
    return _direct_moe(xs, expert_W, valid, dest_chip, dest_row, kept_m)
